# baseline (speedup 1.0000x reference)
import functools

import jax
import jax.numpy as jnp
from jax import lax
from jax.experimental import pallas as pl
from jax.experimental.pallas import tpu as pltpu
from jax.experimental.pallas import tpu_sc as plsc


def kernel(data_batch):
    B, C, H, W = data_batch.shape
    D = min(H, W)
    TOTAL = B * C * H * W
    NC, NS = 2, 16
    NW = NC * NS
    PER_W = TOTAL // NW
    ZW = 32768
    ND = PER_W // ZW
    WIN = 8
    RPW = (B * D) // NW
    PPW = (B * C) // NW
    OWN = C // PPW
    NJ = D // 128

    x1d = data_batch.reshape(TOTAL)
    mesh = plsc.VectorSubcoreMesh(core_axis_name="c", subcore_axis_name="s")

    @functools.partial(
        pl.kernel,
        mesh=mesh,
        out_type=[
            jax.ShapeDtypeStruct((TOTAL,), jnp.float32),
            jax.ShapeDtypeStruct((B * D * 4,), jnp.int32),
        ],
        scratch_types=[
            pltpu.VMEM((ZW,), jnp.float32),
            pltpu.VMEM((NJ, 128), jnp.int32),
            pltpu.VMEM((NJ, 128), jnp.float32),
            pltpu.VMEM((RPW * 4,), jnp.int32),
            pltpu.SemaphoreType.DMA,
            pltpu.SemaphoreType.DMA,
            pltpu.SemaphoreType.DMA,
        ],
    )
    def sc_k(x_hbm, val_hbm, ind_hbm, zbuf, idxb, diagb, indb, zsem, gsem, ssem):
        wid = lax.axis_index("s") * NC + lax.axis_index("c")
        base = wid * PER_W
        lane = lax.broadcasted_iota(jnp.int32, (16,), 0)

        zv = jnp.zeros((16,), jnp.float32)
        for t in range(ZW // 16):
            zbuf[pl.ds(t * 16, 16)] = zv

        handles = []
        for d in range(ND):
            h = pltpu.async_copy(zbuf, val_hbm.at[pl.ds(base + d * ZW, ZW)], zsem)
            handles.append(h)
            if d >= WIN:
                handles[d - WIN].wait()

        bscalar = wid >> 2
        ibase = (wid & 3) * RPW
        bvec = lax.broadcast_in_dim(bscalar, (16,), ())
        ivec = lax.broadcast_in_dim(ibase, (16,), ())
        zero16 = jnp.zeros((16,), jnp.int32)
        for t in range(RPW * 4 // 16):
            e = t * 16 + lane
            k = e >> 2
            col = e & 3
            i = ivec + k
            v = jnp.where(col == 0, bvec, jnp.where(col == 1, zero16, i))
            indb[pl.ds(t * 16, 16)] = v
        pltpu.sync_copy(indb, ind_hbm.at[pl.ds(wid * RPW * 4, RPW * 4)])

        @pl.when((wid & 3) == 0)
        def _():
            bofs = lax.broadcast_in_dim((wid >> 2) * (C * H * W), (16,), ())
            for j in range(NJ):
                for t in range(8):
                    i = j * 128 + t * 16 + lane
                    idxb[j, pl.ds(t * 16, 16)] = bofs + i * (W + 1)
            for j in range(NJ):
                pltpu.async_copy(x_hbm.at[idxb.at[j]], diagb.at[j], gsem).wait()

        for d in range(max(0, ND - WIN), ND):
            handles[d].wait()


        @pl.when((wid & 3) == 0)
        def _():
            for j in range(NJ):
                pltpu.async_copy(diagb.at[j], val_hbm.at[idxb.at[j]], ssem).wait()


    values_1d, indices_1d = sc_k(x1d)
    return (values_1d.reshape(B, C, H, W), indices_1d.reshape(B * D, 4))
